# baseline (speedup 1.0000x reference)
import functools

import jax
import jax.numpy as jnp
from jax import lax
from jax.experimental import pallas as pl
from jax.experimental.pallas import tpu as pltpu
from jax.experimental.pallas import tpu_sc as plsc

E = 320000
D = 128
NCAT = 14
T_ROWS = NCAT * NCAT * NCAT
T_PAD = 2816

NC = 2
NS = 16
NW = NC * NS
R_PER_W = E // NW
GROUP = 400
N_GROUPS = R_PER_W // GROUP
DMA_B = 80
N_DMA = GROUP // DMA_B
JSTEPS = GROUP // 16


def _build_table(e0, e1, e2):
    def body(e0_ref, e1_ref, e2_ref, t_ref):
        t_ref[...] = (
            e0_ref[...][:, None, None, :]
            + e1_ref[...][None, :, None, :]
        ) + e2_ref[...][None, None, :, :]

    t4 = pl.pallas_call(
        body,
        out_shape=jax.ShapeDtypeStruct((NCAT, NCAT, NCAT, D), jnp.float32),
    )(e0, e1, e2)
    return t4.reshape(T_ROWS, D)


def _dg(v, idx):
    return lax.gather(
        v,
        idx[:, None],
        lax.GatherDimensionNumbers(
            offset_dims=(),
            collapsed_slice_dims=(0,),
            start_index_map=(0,),
        ),
        (1,),
        mode=lax.GatherScatterMode.PROMISE_IN_BOUNDS,
    )


_mesh = plsc.VectorSubcoreMesh(core_axis_name="c", subcore_axis_name="s")


@functools.partial(
    pl.kernel,
    mesh=_mesh,
    out_type=jax.ShapeDtypeStruct((E, D), jnp.float32),
    scratch_types=[
        pltpu.VMEM((3 * GROUP,), jnp.int32),
        pltpu.VMEM((3 * GROUP,), jnp.int32),
        pltpu.VMEM((N_DMA, DMA_B), jnp.int32),
        pltpu.VMEM((N_DMA, DMA_B), jnp.int32),
        pltpu.VMEM((GROUP, D), jnp.float32),
        pltpu.VMEM((GROUP, D), jnp.float32),
        pltpu.SemaphoreType.DMA,
        pltpu.SemaphoreType.DMA,
        pltpu.SemaphoreType.DMA,
        pltpu.SemaphoreType.DMA,
        pltpu.SemaphoreType.DMA,
        pltpu.SemaphoreType.DMA,
        pltpu.VMEM_SHARED((T_PAD, D), jnp.float32),
    ],
)
def _sc_lookup(ea_hbm, t_hbm, out_hbm,
               e0b, e1b, c0, c1, r0, r1,
               l0, l1, g0, g1, s0, s1, t_sh):
    sid = lax.axis_index("s")
    wid = sid * NC + lax.axis_index("c")
    base = wid * R_PER_W

    stripe = T_PAD // NS
    pltpu.sync_copy(t_hbm.at[pl.ds(sid * stripe, stripe)],
                    t_sh.at[pl.ds(sid * stripe, stripe)])
    plsc.subcore_barrier()
    ebufs = (e0b, e1b)
    cbufs = (c0, c1)
    rbufs = (r0, r1)
    lsems = (l0, l1)
    gsems = (g0, g1)
    ssems = (s0, s1)

    iota = lax.iota(jnp.int32, 16)
    sel = []
    for o in range(3):
        gpos = iota * 3 + o
        lane = lax.rem(gpos, 16)
        src = lax.div(gpos, 16)
        sel.append((lane, src == 0, src == 1))
    wpat = []
    for i in range(3):
        m = lax.rem(iota + (16 * i), 3)
        wpat.append(jnp.where(m == 0, 196, jnp.where(m == 1, 14, 1)))

    def fire_triples(g):
        p = g % 2
        gbase = base + g * GROUP
        return pltpu.async_copy(
            ea_hbm.at[pl.ds(gbase * 3, 3 * GROUP)], ebufs[p], lsems[p])

    triple_copies = {0: fire_triples(0)}
    store_copies = {}

    for g in range(N_GROUPS):
        p = g % 2
        gbase = base + g * GROUP
        triple_copies.pop(g).wait()
        for j in range(JSTEPS):
            v = [ebufs[p][pl.ds(48 * j + 16 * i, 16)] for i in range(3)]
            w = [v[i] * wpat[i] for i in range(3)]
            c = None
            for lane, is0, is1 in sel:
                t = jnp.where(is0, _dg(w[0], lane),
                              jnp.where(is1, _dg(w[1], lane),
                                        _dg(w[2], lane)))
                c = t if c is None else c + t
            cbufs[p][j // 5, pl.ds((j % 5) * 16, 16)] = c
        if g >= 2:
            store_copies.pop(g - 2).wait()
        gathers = [
            pltpu.async_copy(
                t_sh.at[cbufs[p].at[b]],
                rbufs[p].at[pl.ds(b * DMA_B, DMA_B)],
                gsems[p],
            )
            for b in range(N_DMA)
        ]
        if g + 1 < N_GROUPS:
            triple_copies[g + 1] = fire_triples(g + 1)
        for cp in gathers:
            cp.wait()
        store_copies[g] = pltpu.async_copy(
            rbufs[p], out_hbm.at[pl.ds(gbase, GROUP)], ssems[p])

    for g in (N_GROUPS - 2, N_GROUPS - 1):
        store_copies.pop(g).wait()


def kernel(edge_attr, emb0, emb1, emb2):
    ea = edge_attr.astype(jnp.int32).reshape(-1)
    t = _build_table(emb0, emb1, emb2)
    t = jnp.concatenate([t, jnp.zeros((T_PAD - T_ROWS, D), jnp.float32)])
    return _sc_lookup(ea, t)
